# Initial kernel scaffold; baseline (speedup 1.0000x reference)
import jax
import jax.numpy as jnp
from jax.experimental import pallas as pl


def kernel(xyz_target, xyz_source, feats_target, feats_source, W1, W2):
    raise NotImplementedError("write your pallas kernel here")


# speedup vs baseline: 26.8102x; 26.8102x over previous
import functools
import jax
import jax.numpy as jnp
from jax.experimental import pallas as pl
from jax.experimental.pallas import tpu as pltpu


def _fp_body(xt_ref, xs_ref, ft_ref, fs_ref, w1a_ref, w1b_ref, w2_ref,
             out_ref):
    n_t = xt_ref.shape[2]
    n_s = xs_ref.shape[2]

    d2 = jnp.zeros((n_t, n_s), dtype=jnp.float32)
    for d in range(3):
        tcol = xt_ref[0, d, :].reshape(n_t, 1)
        srow = xs_ref[0, d, :].reshape(1, n_s)
        diff = tcol - srow
        d2 = d2 + diff * diff
    dis = jnp.sqrt(d2)

    s_iota = jax.lax.broadcasted_iota(jnp.int32, (n_t, n_s), 1)

    coeff_mat = jnp.zeros((n_t, n_s), dtype=jnp.float32)
    masked = dis
    nds = []
    onehots = []
    for _ in range(3):
        m = jnp.min(masked, axis=1, keepdims=True)
        idx_cand = jnp.where(masked == m, s_iota, n_s)
        amin = jnp.min(idx_cand, axis=1, keepdims=True)
        onehot = (s_iota == amin)
        nds.append(jnp.maximum(m, 1e-10))
        onehots.append(onehot)
        masked = jnp.where(onehot, jnp.float32(jnp.inf), masked)

    r1, r2, r3 = (1.0 / nd for nd in nds)
    norm = r1 + r2 + r3
    w1_, w2_, w3_ = r1 / norm, r2 / norm, r3 / norm
    wsum = w1_ + w2_ + w3_ + 1e-6
    c1, c2, c3 = w1_ / wsum, w2_ / wsum, w3_ / wsum
    coeff_mat = (jnp.where(onehots[0], c1, 0.0)
                 + jnp.where(onehots[1], c2, 0.0)
                 + jnp.where(onehots[2], c3, 0.0))

    g = jnp.dot(fs_ref[0], w1a_ref[...], preferred_element_type=jnp.float32)
    h = jnp.dot(coeff_mat, g, preferred_element_type=jnp.float32)
    h = h + jnp.dot(ft_ref[0], w1b_ref[...],
                    preferred_element_type=jnp.float32)
    h = jnp.maximum(h, 0.0)
    out = jnp.dot(h, w2_ref[...], preferred_element_type=jnp.float32)
    out_ref[0] = jnp.maximum(out, 0.0)


@jax.jit
def kernel(xyz_target, xyz_source, feats_target, feats_source, W1, W2):
    bs, n_t, _ = xyz_target.shape
    n_s = xyz_source.shape[1]
    c_t = feats_target.shape[2]
    c_s = feats_source.shape[2]

    xt = jnp.transpose(xyz_target, (0, 2, 1))
    xs = jnp.transpose(xyz_source, (0, 2, 1))
    W1a = W1[:c_s]
    W1b = W1[c_s:]

    grid = (bs,)
    out = pl.pallas_call(
        _fp_body,
        grid=grid,
        in_specs=[
            pl.BlockSpec((1, 3, n_t), lambda b: (b, 0, 0)),
            pl.BlockSpec((1, 3, n_s), lambda b: (b, 0, 0)),
            pl.BlockSpec((1, n_t, c_t), lambda b: (b, 0, 0)),
            pl.BlockSpec((1, n_s, c_s), lambda b: (b, 0, 0)),
            pl.BlockSpec((c_s, W1.shape[1]), lambda b: (0, 0)),
            pl.BlockSpec((c_t, W1.shape[1]), lambda b: (0, 0)),
            pl.BlockSpec(W2.shape, lambda b: (0, 0)),
        ],
        out_specs=pl.BlockSpec((1, n_t, W2.shape[1]), lambda b: (b, 0, 0)),
        out_shape=jax.ShapeDtypeStruct((bs, n_t, W2.shape[1]), jnp.float32),
    )(xt, xs, feats_target, feats_source, W1a, W1b, W2)
    return out
